# baseline (speedup 1.0000x reference)
import functools

import jax
import jax.numpy as jnp
from jax import lax
from jax.experimental import pallas as pl
from jax.experimental.pallas import tpu as pltpu
from jax.experimental.pallas import tpu_sc as plsc

_INFO = plsc.get_sparse_core_info()
_NC = _INFO.num_cores
_NS = _INFO.num_subcores
_NW = _NC * _NS
_LANES = _INFO.num_lanes

_CHUNK = 320
_NBUF = 2


def _rsqrt16(x):
    i = plsc.bitcast(x, jnp.int32)
    i = jnp.int32(0x5F3759DF) - lax.shift_right_arithmetic(i, 1)
    y = plsc.bitcast(i, jnp.float32)
    half = x * 0.5
    for _ in range(3):
        y = y * (1.5 - half * y * y)
    return y


def _make_expand(n_tokens, n_rows, d):
    assert n_tokens % (_NW * _CHUNK) == 0
    per_w = n_tokens // _NW
    n_chunks = per_w // _CHUNK
    n_col = d // _LANES
    mesh = plsc.VectorSubcoreMesh(core_axis_name="c", subcore_axis_name="s")

    @functools.partial(
        pl.kernel,
        out_type=jax.ShapeDtypeStruct((n_tokens, d), jnp.float32),
        mesh=mesh,
        compiler_params=pltpu.CompilerParams(needs_layout_passes=False),
        scratch_types=[
            pltpu.VMEM_SHARED((_NS * per_w,), jnp.int32),
            pltpu.VMEM((n_rows, d), jnp.float32),
            pltpu.VMEM((d,), jnp.float32),
            pltpu.VMEM((d,), jnp.float32),
            pltpu.VMEM((_NBUF, _CHUNK, d), jnp.float32),
            pltpu.SMEM((_NBUF * _CHUNK,), jnp.int32),
            pltpu.SemaphoreType.DMA,
            pltpu.SemaphoreType.DMA,
            pltpu.SemaphoreType.DMA,
            pltpu.SemaphoreType.DMA,
            pltpu.SemaphoreType.DMA,
        ],
    )
    def expand_kernel(idx_hbm, tab_hbm, gamma_hbm, beta_hbm, out_hbm,
                      idx_v, tab_v, gamma_v, beta_v, rows_v, idx_sm,
                      semx, semi0, semi1, sem0, sem1):
        sid = lax.axis_index("s")
        wid = sid * _NC + lax.axis_index("c")
        pltpu.async_copy(
            idx_hbm.at[wid], idx_v.at[pl.ds(sid * per_w, per_w)], semx
        )
        pltpu.sync_copy(tab_hbm, tab_v)
        pltpu.sync_copy(gamma_hbm, gamma_v)
        pltpu.sync_copy(beta_hbm, beta_v)

        inv_d = jnp.float32(1.0 / d)
        for v in range(n_rows):
            acc = jnp.zeros((_LANES,), jnp.float32)
            acc2 = jnp.zeros((_LANES,), jnp.float32)
            for c in range(n_col):
                x = tab_v[v, pl.ds(c * _LANES, _LANES)]
                acc = acc + x
                acc2 = acc2 + x * x
            s1 = jnp.sum(acc)
            s2 = jnp.sum(acc2)
            mean = s1 * inv_d
            var = s2 * inv_d - mean * mean
            mean_v = lax.broadcast(mean, (_LANES,))
            scale_v = _rsqrt16(lax.broadcast(var + 1e-12, (_LANES,)))
            for c in range(n_col):
                sl = pl.ds(c * _LANES, _LANES)
                tab_v[v, sl] = (
                    (tab_v[v, sl] - mean_v) * scale_v * gamma_v[sl] + beta_v[sl]
                )

        pltpu.make_async_copy(
            idx_hbm.at[0], idx_v.at[pl.ds(0, per_w)], semx
        ).wait()

        base = wid * per_w
        sems = [sem0, sem1]
        semis = [semi0, semi1]

        def fire_idx(k):
            pltpu.async_copy(
                idx_v.at[pl.ds(sid * per_w + k * _CHUNK, _CHUNK)],
                idx_sm.at[pl.ds((k % _NBUF) * _CHUNK, _CHUNK)],
                semis[k % _NBUF],
            )

        def wait_idx(k):
            pltpu.make_async_copy(
                idx_v.at[pl.ds(0, _CHUNK)],
                idx_sm.at[pl.ds((k % _NBUF) * _CHUNK, _CHUNK)],
                semis[k % _NBUF],
            ).wait()

        def build(k, buf):
            @plsc.parallel_loop(0, _CHUNK, unroll=8)
            def _(t):
                v = idx_sm[buf * _CHUNK + t]
                for c in range(n_col):
                    sl = pl.ds(c * _LANES, _LANES)
                    rows_v[buf, t, sl] = tab_v[v, sl]

        def store(k, buf):
            pltpu.async_copy(
                rows_v.at[buf],
                out_hbm.at[pl.ds(base + k * _CHUNK, _CHUNK)],
                sems[buf],
            )

        def drain_store(buf):
            pltpu.make_async_copy(
                rows_v.at[buf],
                out_hbm.at[pl.ds(0, _CHUNK)],
                sems[buf],
            ).wait()

        fire_idx(0)
        for k in range(n_chunks):
            buf = k % _NBUF
            wait_idx(k)
            if k + 1 < n_chunks:
                fire_idx(k + 1)
            if k >= _NBUF:
                drain_store(buf)
            build(k, buf)
            store(k, buf)
        for buf in range(_NBUF):
            drain_store(buf)

    return expand_kernel


def kernel(snp, is_padding, snp_table, ln_gamma, ln_beta):
    b, l = snp.shape
    v, d = snp_table.shape
    n = b * l
    idx = snp.reshape(_NW, n // _NW).astype(jnp.int32)
    out = _make_expand(n, v, d)(idx, snp_table, ln_gamma, ln_beta)
    return out.reshape(b, l, d), is_padding
